# baseline (speedup 1.0000x reference)
import jax
import jax.numpy as jnp
from jax.experimental import pallas as pl

N_USERS = 4096
N_POSTS = 4096
N_ALL = N_USERS + N_POSTS
H = 128

_INTERPRET = False


def _proj_kernel(x_ref, w_ref, b_ref, wg_ref, out_ref):
    emb = jnp.dot(x_ref[...], w_ref[0], preferred_element_type=jnp.float32)
    emb = emb + b_ref[0]
    out_ref[...] = jnp.dot(emb, wg_ref[...], preferred_element_type=jnp.float32)


def _layer1_kernel(adj_ref, s_ref, b_ref, w_ref, out_ref):
    acc = jnp.dot(adj_ref[...], s_ref[...], preferred_element_type=jnp.float32)
    h = jnp.maximum(acc + b_ref[...], 0.0)
    out_ref[...] = jnp.dot(h, w_ref[...], preferred_element_type=jnp.float32)


def _layer2_kernel(adj_ref, s_ref, b_ref, out_ref):
    acc = jnp.dot(adj_ref[...], s_ref[...], preferred_element_type=jnp.float32)
    out_ref[...] = jnp.maximum(acc + b_ref[...], 0.0)


def _head_kernel(uf_ref, pf_ref, ui_ref, pi_ref, wh0_ref, bh0_ref,
                 wh1_ref, bh1_ref, wh2_ref, bh2_ref, out_ref):
    bb = ui_ref.shape[1]
    ui = ui_ref[0]
    pi = pi_ref[0]
    iota_u = jax.lax.broadcasted_iota(jnp.int32, (bb, N_USERS), 1)
    oh_u = (iota_u == ui).astype(jnp.float32)
    bu_emb = jnp.dot(oh_u, uf_ref[...], preferred_element_type=jnp.float32)
    iota_p = jax.lax.broadcasted_iota(jnp.int32, (bb, N_POSTS), 1)
    oh_p = (iota_p == pi).astype(jnp.float32)
    bp_emb = jnp.dot(oh_p, pf_ref[...], preferred_element_type=jnp.float32)
    combined = jnp.concatenate([bu_emb, bp_emb], axis=1)
    x = jnp.maximum(
        jnp.dot(combined, wh0_ref[...], preferred_element_type=jnp.float32)
        + bh0_ref[...], 0.0)
    x = jnp.maximum(
        jnp.dot(x, wh1_ref[...], preferred_element_type=jnp.float32)
        + bh1_ref[...], 0.0)
    s = jnp.dot(x, wh2_ref[...], preferred_element_type=jnp.float32) + bh2_ref[...]
    out_ref[...] = jax.nn.sigmoid(s)


def kernel(user_features, post_features, adj_matrix, user_indices, post_indices,
           Wu, bu, Wp, bp, Wg0, bg0, Wg1, bg1, Wh0, bh0, Wh1, bh1, Wh2, bh2):
    f32 = jnp.float32
    x = jnp.concatenate([user_features, post_features], axis=0)
    Wproj = jnp.stack([Wu, Wp])
    bproj = jnp.stack([bu, bp]).reshape(2, 1, H)

    PB = 512
    support0 = pl.pallas_call(
        _proj_kernel,
        grid=(N_ALL // PB,),
        in_specs=[
            pl.BlockSpec((PB, x.shape[1]), lambda i: (i, 0)),
            pl.BlockSpec((1, Wproj.shape[1], H), lambda i: (i // (N_USERS // PB), 0, 0)),
            pl.BlockSpec((1, 1, H), lambda i: (i // (N_USERS // PB), 0, 0)),
            pl.BlockSpec((H, H), lambda i: (0, 0)),
        ],
        out_specs=pl.BlockSpec((PB, H), lambda i: (i, 0)),
        out_shape=jax.ShapeDtypeStruct((N_ALL, H), f32),
        interpret=_INTERPRET,
    )(x, Wproj, bproj, Wg0)

    RM = 256
    layer_specs = dict(
        grid=(N_ALL // RM,),
        out_specs=pl.BlockSpec((RM, H), lambda i: (i, 0)),
        out_shape=jax.ShapeDtypeStruct((N_ALL, H), f32),
        interpret=_INTERPRET,
    )
    support1 = pl.pallas_call(
        _layer1_kernel,
        in_specs=[
            pl.BlockSpec((RM, N_ALL), lambda i: (i, 0)),
            pl.BlockSpec((N_ALL, H), lambda i: (0, 0)),
            pl.BlockSpec((1, H), lambda i: (0, 0)),
            pl.BlockSpec((H, H), lambda i: (0, 0)),
        ],
        **layer_specs,
    )(adj_matrix, support0, bg0.reshape(1, H), Wg1)

    h2 = pl.pallas_call(
        _layer2_kernel,
        in_specs=[
            pl.BlockSpec((RM, N_ALL), lambda i: (i, 0)),
            pl.BlockSpec((N_ALL, H), lambda i: (0, 0)),
            pl.BlockSpec((1, H), lambda i: (0, 0)),
        ],
        **layer_specs,
    )(adj_matrix, support1, bg1.reshape(1, H))

    user_final = h2[:N_USERS]
    post_final = h2[N_USERS:]

    BB = 256
    nb = user_indices.shape[0] // BB
    ui = user_indices.astype(jnp.int32).reshape(nb, BB, 1)
    pi = post_indices.astype(jnp.int32).reshape(nb, BB, 1)
    scores = pl.pallas_call(
        _head_kernel,
        grid=(nb,),
        in_specs=[
            pl.BlockSpec((N_USERS, H), lambda i: (0, 0)),
            pl.BlockSpec((N_POSTS, H), lambda i: (0, 0)),
            pl.BlockSpec((1, BB, 1), lambda i: (i, 0, 0)),
            pl.BlockSpec((1, BB, 1), lambda i: (i, 0, 0)),
            pl.BlockSpec((2 * H, H), lambda i: (0, 0)),
            pl.BlockSpec((1, H), lambda i: (0, 0)),
            pl.BlockSpec((H, H // 2), lambda i: (0, 0)),
            pl.BlockSpec((1, H // 2), lambda i: (0, 0)),
            pl.BlockSpec((H // 2, 1), lambda i: (0, 0)),
            pl.BlockSpec((1, 1), lambda i: (0, 0)),
        ],
        out_specs=pl.BlockSpec((BB, 1), lambda i: (i, 0)),
        out_shape=jax.ShapeDtypeStruct((user_indices.shape[0], 1), f32),
        interpret=_INTERPRET,
    )(user_final, post_final, ui, pi, Wh0, bh0.reshape(1, H),
      Wh1, bh1.reshape(1, H // 2), Wh2, bh2.reshape(1, 1))
    return jnp.squeeze(scores, axis=-1)


# speedup vs baseline: 1.0415x; 1.0415x over previous
import functools

import jax
import jax.numpy as jnp
from jax import lax
from jax.experimental import pallas as pl
from jax.experimental.pallas import tpu as pltpu
from jax.experimental.pallas import tpu_sc as plsc

N_USERS = 4096
N_POSTS = 4096
N_ALL = N_USERS + N_POSTS
H = 128

_SC_CORES = 2
_SC_SUBCORES = 16
_SC_WORKERS = _SC_CORES * _SC_SUBCORES
_GCHUNK = 128

_INTERPRET = False


def _proj_kernel(x_ref, w_ref, b_ref, wg_ref, out_ref):
    emb = jnp.dot(x_ref[...], w_ref[0], preferred_element_type=jnp.float32)
    emb = emb + b_ref[0]
    out_ref[...] = jnp.dot(emb, wg_ref[...], preferred_element_type=jnp.float32)


def _layer1_kernel(adj_ref, s_ref, b_ref, w_ref, out_ref):
    acc = jnp.dot(adj_ref[...], s_ref[...], preferred_element_type=jnp.float32)
    h = jnp.maximum(acc + b_ref[...], 0.0)
    out_ref[...] = jnp.dot(h, w_ref[...], preferred_element_type=jnp.float32)


def _layer2_kernel(adj_ref, s_ref, b_ref, out_ref):
    acc = jnp.dot(adj_ref[...], s_ref[...], preferred_element_type=jnp.float32)
    out_ref[...] = jnp.maximum(acc + b_ref[...], 0.0)


def _gather_body(table_hbm, idx_hbm, out_hbm, idx_v, rows_v, sem):
    wid = lax.axis_index("s") * _SC_CORES + lax.axis_index("c")
    rows_per_worker = (2 * N_USERS) // _SC_WORKERS
    for j in range(rows_per_worker // _GCHUNK):
        base = wid * rows_per_worker + j * _GCHUNK
        pltpu.sync_copy(idx_hbm.at[pl.ds(base, _GCHUNK)], idx_v)
        pltpu.async_copy(table_hbm.at[idx_v], rows_v, sem).wait()
        pltpu.sync_copy(rows_v, out_hbm.at[pl.ds(base, _GCHUNK)])


_gather_rows = functools.partial(
    pl.kernel,
    out_type=jax.ShapeDtypeStruct((2 * N_USERS, H), jnp.float32),
    scratch_types=[
        pltpu.VMEM((_GCHUNK,), jnp.int32),
        pltpu.VMEM((_GCHUNK, H), jnp.float32),
        pltpu.SemaphoreType.DMA,
    ],
    mesh=plsc.VectorSubcoreMesh(core_axis_name="c", subcore_axis_name="s"),
)(_gather_body)


def _head_kernel(bu_ref, bp_ref, w0u_ref, w0p_ref, b0_ref, w1_ref, b1_ref,
                 w2_ref, b2_ref, out_ref):
    x = (jnp.dot(bu_ref[...], w0u_ref[...], preferred_element_type=jnp.float32)
         + jnp.dot(bp_ref[...], w0p_ref[...], preferred_element_type=jnp.float32)
         + b0_ref[...])
    x = jnp.maximum(x, 0.0)
    x = jnp.maximum(
        jnp.dot(x, w1_ref[...], preferred_element_type=jnp.float32) + b1_ref[...],
        0.0)
    s = jnp.dot(x, w2_ref[...], preferred_element_type=jnp.float32) + b2_ref[...]
    out_ref[...] = jax.nn.sigmoid(s)


def kernel(user_features, post_features, adj_matrix, user_indices, post_indices,
           Wu, bu, Wp, bp, Wg0, bg0, Wg1, bg1, Wh0, bh0, Wh1, bh1, Wh2, bh2):
    f32 = jnp.float32
    x = jnp.concatenate([user_features, post_features], axis=0)
    Wproj = jnp.stack([Wu, Wp])
    bproj = jnp.stack([bu, bp]).reshape(2, 1, H)

    PB = 512
    support0 = pl.pallas_call(
        _proj_kernel,
        grid=(N_ALL // PB,),
        in_specs=[
            pl.BlockSpec((PB, x.shape[1]), lambda i: (i, 0)),
            pl.BlockSpec((1, Wproj.shape[1], H), lambda i: (i // (N_USERS // PB), 0, 0)),
            pl.BlockSpec((1, 1, H), lambda i: (i // (N_USERS // PB), 0, 0)),
            pl.BlockSpec((H, H), lambda i: (0, 0)),
        ],
        out_specs=pl.BlockSpec((PB, H), lambda i: (i, 0)),
        out_shape=jax.ShapeDtypeStruct((N_ALL, H), f32),
        interpret=_INTERPRET,
    )(x, Wproj, bproj, Wg0)

    RM = 512
    layer_specs = dict(
        grid=(N_ALL // RM,),
        out_specs=pl.BlockSpec((RM, H), lambda i: (i, 0)),
        out_shape=jax.ShapeDtypeStruct((N_ALL, H), f32),
        interpret=_INTERPRET,
    )
    support1 = pl.pallas_call(
        _layer1_kernel,
        in_specs=[
            pl.BlockSpec((RM, N_ALL), lambda i: (i, 0)),
            pl.BlockSpec((N_ALL, H), lambda i: (0, 0)),
            pl.BlockSpec((1, H), lambda i: (0, 0)),
            pl.BlockSpec((H, H), lambda i: (0, 0)),
        ],
        **layer_specs,
    )(adj_matrix, support0, bg0.reshape(1, H), Wg1)

    h2 = pl.pallas_call(
        _layer2_kernel,
        in_specs=[
            pl.BlockSpec((RM, N_ALL), lambda i: (i, 0)),
            pl.BlockSpec((N_ALL, H), lambda i: (0, 0)),
            pl.BlockSpec((1, H), lambda i: (0, 0)),
        ],
        **layer_specs,
    )(adj_matrix, support1, bg1.reshape(1, H))

    all_idx = jnp.concatenate([
        user_indices.astype(jnp.int32),
        post_indices.astype(jnp.int32) + N_USERS,
    ])
    gathered = _gather_rows(h2, all_idx)

    BB = 512
    nb = N_USERS // BB
    scores = pl.pallas_call(
        _head_kernel,
        grid=(nb,),
        in_specs=[
            pl.BlockSpec((BB, H), lambda i: (i, 0)),
            pl.BlockSpec((BB, H), lambda i: (i + nb, 0)),
            pl.BlockSpec((H, H), lambda i: (0, 0)),
            pl.BlockSpec((H, H), lambda i: (0, 0)),
            pl.BlockSpec((1, H), lambda i: (0, 0)),
            pl.BlockSpec((H, H // 2), lambda i: (0, 0)),
            pl.BlockSpec((1, H // 2), lambda i: (0, 0)),
            pl.BlockSpec((H // 2, 1), lambda i: (0, 0)),
            pl.BlockSpec((1, 1), lambda i: (0, 0)),
        ],
        out_specs=pl.BlockSpec((BB, 1), lambda i: (i, 0)),
        out_shape=jax.ShapeDtypeStruct((N_USERS, 1), f32),
        interpret=_INTERPRET,
    )(gathered, gathered, Wh0[:H], Wh0[H:], bh0.reshape(1, H),
      Wh1, bh1.reshape(1, H // 2), Wh2, bh2.reshape(1, 1))
    return jnp.squeeze(scores, axis=-1)


# speedup vs baseline: 1.2062x; 1.1581x over previous
import jax
import jax.numpy as jnp
from jax.experimental import pallas as pl
from jax.experimental.pallas import tpu as pltpu

N_USERS = 4096
N_POSTS = 4096
N_ALL = N_USERS + N_POSTS
BATCH = 4096
H = 128

_RM = 256
_NB = N_ALL // _RM
_NBU = N_USERS // _RM

_INTERPRET = False


def _gnn_kernel(uf_ref, pf_ref, adj_ref, ui_ref, pi_ref,
                wu_ref, bu_ref, wp_ref, bp_ref,
                wg0_ref, bg0_ref, wg1_ref, bg1_ref,
                w0u_ref, w0p_ref, b0_ref, w1_ref, b1_ref, w2_ref, b2_ref,
                out_ref, s0_ref, s1_ref, cu_ref, cp_ref):
    f32 = jnp.float32
    i = pl.program_id(0)

    @pl.when(i == 0)
    def _proj():
        for h, (f_ref, w_ref, b_ref) in enumerate(
                ((uf_ref, wu_ref, bu_ref), (pf_ref, wp_ref, bp_ref))):
            emb = jnp.dot(f_ref[...], w_ref[...],
                          preferred_element_type=f32) + b_ref[...]
            s0_ref[pl.ds(h * N_USERS, N_USERS), :] = jnp.dot(
                emb, wg0_ref[...], preferred_element_type=f32)
        cu_ref[...] = jnp.zeros(cu_ref.shape, f32)
        cp_ref[...] = jnp.zeros(cp_ref.shape, f32)

    @pl.when((i >= 1) & (i <= _NB))
    def _l1():
        acc = jnp.dot(adj_ref[...], s0_ref[...], preferred_element_type=f32)
        h1 = jnp.maximum(acc + bg0_ref[...], 0.0)
        s1_ref[pl.ds((i - 1) * _RM, _RM), :] = jnp.dot(
            h1, wg1_ref[...], preferred_element_type=f32)

    @pl.when((i > _NB) & (i <= 2 * _NB))
    def _l2():
        b_loc = i - _NB - 1
        acc = jnp.dot(adj_ref[...], s1_ref[...], preferred_element_type=f32)
        h2_blk = jnp.maximum(acc + bg1_ref[...], 0.0)
        iota = jax.lax.broadcasted_iota(jnp.int32, (BATCH, _RM), 1)

        @pl.when(b_loc < _NBU)
        def _users():
            oh = (ui_ref[...] == iota + b_loc * _RM).astype(f32)
            cu_ref[...] += jnp.dot(oh, h2_blk, preferred_element_type=f32)

        @pl.when(b_loc >= _NBU)
        def _posts():
            oh = (pi_ref[...] == iota + (b_loc * _RM - N_USERS)).astype(f32)
            cp_ref[...] += jnp.dot(oh, h2_blk, preferred_element_type=f32)

    @pl.when(i == 2 * _NB + 1)
    def _head():
        x = (jnp.dot(cu_ref[...], w0u_ref[...], preferred_element_type=f32)
             + jnp.dot(cp_ref[...], w0p_ref[...], preferred_element_type=f32)
             + b0_ref[...])
        x = jnp.maximum(x, 0.0)
        x = jnp.maximum(
            jnp.dot(x, w1_ref[...], preferred_element_type=f32) + b1_ref[...],
            0.0)
        s = jnp.dot(x, w2_ref[...], preferred_element_type=f32) + b2_ref[...]
        out_ref[...] = jax.nn.sigmoid(s)


def kernel(user_features, post_features, adj_matrix, user_indices, post_indices,
           Wu, bu, Wp, bp, Wg0, bg0, Wg1, bg1, Wh0, bh0, Wh1, bh1, Wh2, bh2):
    f32 = jnp.float32
    d_in = user_features.shape[1]
    adj_map = lambda i: (
        jnp.where(i > _NB, jnp.minimum(i - _NB - 1, _NB - 1),
                  jnp.maximum(i - 1, 0)), 0)
    const2 = lambda i: (0, 0)
    scores = pl.pallas_call(
        _gnn_kernel,
        grid=(2 * _NB + 2,),
        in_specs=[
            pl.BlockSpec((N_USERS, d_in), const2),
            pl.BlockSpec((N_POSTS, d_in), const2),
            pl.BlockSpec((_RM, N_ALL), adj_map),
            pl.BlockSpec((BATCH, 1), const2),
            pl.BlockSpec((BATCH, 1), const2),
            pl.BlockSpec((d_in, H), const2),
            pl.BlockSpec((1, H), const2),
            pl.BlockSpec((d_in, H), const2),
            pl.BlockSpec((1, H), const2),
            pl.BlockSpec((H, H), const2),
            pl.BlockSpec((1, H), const2),
            pl.BlockSpec((H, H), const2),
            pl.BlockSpec((1, H), const2),
            pl.BlockSpec((H, H), const2),
            pl.BlockSpec((H, H), const2),
            pl.BlockSpec((1, H), const2),
            pl.BlockSpec((H, H // 2), const2),
            pl.BlockSpec((1, H // 2), const2),
            pl.BlockSpec((H // 2, 1), const2),
            pl.BlockSpec((1, 1), const2),
        ],
        out_specs=pl.BlockSpec((BATCH, 1), const2),
        out_shape=jax.ShapeDtypeStruct((BATCH, 1), f32),
        scratch_shapes=[
            pltpu.VMEM((N_ALL, H), f32),
            pltpu.VMEM((N_ALL, H), f32),
            pltpu.VMEM((BATCH, H), f32),
            pltpu.VMEM((BATCH, H), f32),
        ],
        interpret=_INTERPRET,
    )(user_features, post_features, adj_matrix,
      user_indices.astype(jnp.int32).reshape(BATCH, 1),
      post_indices.astype(jnp.int32).reshape(BATCH, 1),
      Wu, bu.reshape(1, H), Wp, bp.reshape(1, H),
      Wg0, bg0.reshape(1, H), Wg1, bg1.reshape(1, H),
      Wh0[:H], Wh0[H:], bh0.reshape(1, H),
      Wh1, bh1.reshape(1, H // 2), Wh2, bh2.reshape(1, 1))
    return jnp.squeeze(scores, axis=-1)
